# baseline (speedup 1.0000x reference)
import jax
import jax.numpy as jnp
from jax.experimental import pallas as pl
from jax.experimental.pallas import tpu as pltpu

_NUM_CLASSES = 599
_OUT_HW = 40
_IMG = 1024.0
_CB = 128


def _kern(xf_ref, boxes_ref, cls_ref, rh_ref, rw_ref, out_ref,
          wf_ref, oht_ref, inv_ref):
    j = pl.program_id(1)
    n = boxes_ref.shape[1]

    @pl.when(j == 0)
    def _prep():
        bx = boxes_ref[0]
        cls = cls_ref[0]
        s = jnp.float32(_OUT_HW / _IMG)
        rb = jnp.round(bx * s)
        x1 = jnp.maximum(rb[:, 0:1], 0.0)
        y1 = jnp.maximum(rb[:, 1:2], 0.0)
        x2 = jnp.minimum(rb[:, 2:3], float(_OUT_HW))
        y2 = jnp.minimum(rb[:, 3:4], float(_OUT_HW))
        g = jax.lax.broadcasted_iota(jnp.int32, (n, _OUT_HW), 1).astype(jnp.float32)
        my = ((g >= y1) & (g < y2)).astype(jnp.float32)
        mx = ((g >= x1) & (g < x2)).astype(jnp.float32)
        cy = jnp.sum(my, axis=1, keepdims=True)
        cx = jnp.sum(mx, axis=1, keepdims=True)
        vf = ((x1 < x2) & (y1 < y2)).astype(jnp.float32)
        scale_n = vf / jnp.maximum(cy * cx, 1.0)
        wy = jnp.dot(my, rh_ref[...], preferred_element_type=jnp.float32)
        wx = jnp.dot(mx, rw_ref[...], preferred_element_type=jnp.float32)
        wf_ref[...] = wy * wx * scale_n
        ki = jax.lax.broadcasted_iota(jnp.int32, (_NUM_CLASSES, n), 0)
        oht = (ki == cls).astype(jnp.float32)
        oht_ref[...] = oht
        counts = jnp.dot(oht, vf, preferred_element_type=jnp.float32)
        inv_ref[...] = 1.0 / jnp.maximum(counts, 1.0)

    pooled = jax.lax.dot_general(
        wf_ref[...], xf_ref[0], (((1,), (1,)), ((), ())),
        preferred_element_type=jnp.float32)
    sums = jnp.dot(oht_ref[...], pooled,
                   preferred_element_type=jnp.float32)
    out_ref[0] = sums * inv_ref[...]


def kernel(features, gt_boxes, gt_classes):
    B, C, H, W = features.shape
    N = gt_boxes.shape[1]
    HW = H * W
    nj = C // _CB

    R = jax.image.resize(jnp.eye(H, dtype=jnp.float32), (_OUT_HW, H),
                         method='bilinear', antialias=False)
    RH = jnp.repeat(R, W, axis=1)
    RW = jnp.tile(R, (1, W))

    xf = features.reshape(B, C, HW)
    clsr = gt_classes.astype(jnp.int32).reshape(B, 1, N)

    out = pl.pallas_call(
        _kern,
        grid=(B, nj),
        in_specs=[
            pl.BlockSpec((1, _CB, HW), lambda b, j: (b, j, 0)),
            pl.BlockSpec((1, N, 4), lambda b, j: (b, 0, 0)),
            pl.BlockSpec((1, 1, N), lambda b, j: (b, 0, 0)),
            pl.BlockSpec((_OUT_HW, HW), lambda b, j: (0, 0)),
            pl.BlockSpec((_OUT_HW, HW), lambda b, j: (0, 0)),
        ],
        out_specs=pl.BlockSpec((1, _NUM_CLASSES, _CB), lambda b, j: (b, 0, j)),
        out_shape=jax.ShapeDtypeStruct((B, _NUM_CLASSES, C), jnp.float32),
        scratch_shapes=[
            pltpu.VMEM((N, HW), jnp.float32),
            pltpu.VMEM((_NUM_CLASSES, N), jnp.float32),
            pltpu.VMEM((_NUM_CLASSES, 1), jnp.float32),
        ],
    )(xf, gt_boxes, clsr, RH, RW)
    return out
